# baseline (speedup 1.0000x reference)
import functools

import jax
import jax.numpy as jnp
from jax import lax
from jax.experimental import pallas as pl
from jax.experimental.pallas import tpu as pltpu
from jax.experimental.pallas import tpu_sc as plsc

NC, NS, L = 2, 16, 16
NW = NC * NS
N_KP = 16
ROWS, COLS = 16384, 100
ROWS_PER_TILE = ROWS // NW
RCHUNK = 64
NCHUNK = ROWS_PER_TILE // RCHUNK
COL_OFFS = (0, 16, 32, 48, 64, 80, 84)
MAGIC = 8388607.5


def _vgather(vec, idx):
    dn = lax.GatherDimensionNumbers(
        offset_dims=(), collapsed_slice_dims=(0,), start_index_map=(0,)
    )
    return lax.gather(
        vec, idx[:, None], dn, slice_sizes=(1,),
        mode=lax.GatherScatterMode.PROMISE_IN_BOUNDS,
    )


_mesh = plsc.VectorSubcoreMesh(
    core_axis_name="c", subcore_axis_name="s", num_cores=NC, num_subcores=NS
)


@functools.partial(
    pl.kernel,
    out_type=jax.ShapeDtypeStruct((ROWS, COLS), jnp.float32),
    mesh=_mesh,
    compiler_params=pltpu.CompilerParams(use_tc_tiling_on_sc=True),
    scratch_types=[
        pltpu.VMEM((RCHUNK, COLS), jnp.float32),
        pltpu.VMEM((RCHUNK, COLS), jnp.float32),
        pltpu.VMEM((RCHUNK, COLS), jnp.float32),
        pltpu.VMEM((RCHUNK, COLS), jnp.float32),
        pltpu.VMEM((N_KP,), jnp.float32),
        pltpu.SemaphoreType.DMA,
        pltpu.SemaphoreType.DMA,
        pltpu.SemaphoreType.DMA,
        pltpu.SemaphoreType.DMA,
    ],
)
def _calib(x_hbm, kp_hbm, out_hbm,
           x_v0, x_v1, y_v0, y_v1, kp_v,
           in_sem0, in_sem1, out_sem0, out_sem1):
    x_bufs = (x_v0, x_v1)
    y_bufs = (y_v0, y_v1)
    in_sems = (in_sem0, in_sem1)
    out_sems = (out_sem0, out_sem1)
    wid = lax.axis_index("s") * NC + lax.axis_index("c")
    base_row = wid * ROWS_PER_TILE

    pltpu.sync_copy(kp_hbm, kp_v)

    lane = lax.iota(jnp.int32, L)
    lane_l = jnp.maximum(lane - 1, 0)
    raw = kp_v[...]
    y_r = 1.0 / (1.0 + jnp.exp(-raw))
    y_l = _vgather(y_r, lane_l)
    x_r = lane.astype(jnp.float32) * (1.0 / 15.0)
    x_l = lane_l.astype(jnp.float32) * (1.0 / 15.0)
    b_seg = (y_r - y_l) / (x_r - x_l + 1e-8)
    a_seg = y_l - x_l * b_seg
    shift = jnp.minimum(lane + 1, 15)
    tab_b_vec = _vgather(b_seg, shift) * (1.0 / 15.0)
    tab_a_vec = _vgather(a_seg, shift)

    in_dma = [None] * NCHUNK
    out_dma = [None] * NCHUNK
    in_dma[0] = pltpu.async_copy(
        x_hbm.at[pl.ds(base_row, RCHUNK), :], x_bufs[0], in_sems[0])
    for k in range(NCHUNK):
        cur = k % 2
        r0 = base_row + k * RCHUNK
        if k + 1 < NCHUNK:
            in_dma[k + 1] = pltpu.async_copy(
                x_hbm.at[pl.ds(r0 + RCHUNK, RCHUNK), :],
                x_bufs[1 - cur], in_sems[1 - cur])
        in_dma[k].wait()
        if k >= 2:
            out_dma[k - 2].wait()
        x_v = x_bufs[cur]
        y_v = y_bufs[cur]

        @plsc.parallel_loop(0, RCHUNK, step=1, unroll=4)
        def _body(r):
            for c in COL_OFFS:
                v = x_v[r, pl.ds(c, L)]
                u = jnp.minimum(jnp.maximum(v * 15.0, 0.0), 15.0)
                idx = jnp.minimum(u, 14.0).astype(jnp.int32)
                av = _vgather(tab_a_vec, idx)
                bv = _vgather(tab_b_vec, idx)
                y_v[r, pl.ds(c, L)] = av + u * bv

        out_dma[k] = pltpu.async_copy(
            y_v, out_hbm.at[pl.ds(r0, RCHUNK), :], out_sems[cur])
    out_dma[NCHUNK - 2].wait()
    out_dma[NCHUNK - 1].wait()


def kernel(x, keypoint_y):
    return _calib(x, keypoint_y)


# speedup vs baseline: 1.0302x; 1.0302x over previous
import functools

import jax
import jax.numpy as jnp
from jax import lax
from jax.experimental import pallas as pl
from jax.experimental.pallas import tpu as pltpu
from jax.experimental.pallas import tpu_sc as plsc

NC, NS, L = 2, 16, 16
NW = NC * NS
N_KP = 16
ROWS, COLS = 16384, 100
ROWS_PER_TILE = ROWS // NW
RCHUNK = 128
NCHUNK = ROWS_PER_TILE // RCHUNK
COL_OFFS = (0, 16, 32, 48, 64, 80, 84)
MAGIC = 8388607.5


def _vgather(vec, idx):
    dn = lax.GatherDimensionNumbers(
        offset_dims=(), collapsed_slice_dims=(0,), start_index_map=(0,)
    )
    return lax.gather(
        vec, idx[:, None], dn, slice_sizes=(1,),
        mode=lax.GatherScatterMode.PROMISE_IN_BOUNDS,
    )


_mesh = plsc.VectorSubcoreMesh(
    core_axis_name="c", subcore_axis_name="s", num_cores=NC, num_subcores=NS
)


@functools.partial(
    pl.kernel,
    out_type=jax.ShapeDtypeStruct((ROWS, COLS), jnp.float32),
    mesh=_mesh,
    compiler_params=pltpu.CompilerParams(use_tc_tiling_on_sc=True),
    scratch_types=[
        pltpu.VMEM((RCHUNK, COLS), jnp.float32),
        pltpu.VMEM((RCHUNK, COLS), jnp.float32),
        pltpu.VMEM((RCHUNK, COLS), jnp.float32),
        pltpu.VMEM((RCHUNK, COLS), jnp.float32),
        pltpu.VMEM((N_KP,), jnp.float32),
        pltpu.SemaphoreType.DMA,
        pltpu.SemaphoreType.DMA,
        pltpu.SemaphoreType.DMA,
        pltpu.SemaphoreType.DMA,
    ],
)
def _calib(x_hbm, kp_hbm, out_hbm,
           x_v0, x_v1, y_v0, y_v1, kp_v,
           in_sem0, in_sem1, out_sem0, out_sem1):
    x_bufs = (x_v0, x_v1)
    y_bufs = (y_v0, y_v1)
    in_sems = (in_sem0, in_sem1)
    out_sems = (out_sem0, out_sem1)
    wid = lax.axis_index("s") * NC + lax.axis_index("c")
    base_row = wid * ROWS_PER_TILE

    pltpu.sync_copy(kp_hbm, kp_v)

    lane = lax.iota(jnp.int32, L)
    lane_l = jnp.maximum(lane - 1, 0)
    raw = kp_v[...]
    y_r = 1.0 / (1.0 + jnp.exp(-raw))
    y_l = _vgather(y_r, lane_l)
    x_r = lane.astype(jnp.float32) * (1.0 / 15.0)
    x_l = lane_l.astype(jnp.float32) * (1.0 / 15.0)
    b_seg = (y_r - y_l) / (x_r - x_l + 1e-8)
    a_seg = y_l - x_l * b_seg
    shift = jnp.minimum(lane + 1, 15)
    tab_b_vec = _vgather(b_seg, shift) * (1.0 / 15.0)
    tab_a_vec = _vgather(a_seg, shift)

    in_dma = [None] * NCHUNK
    out_dma = [None] * NCHUNK
    in_dma[0] = pltpu.async_copy(
        x_hbm.at[pl.ds(base_row, RCHUNK), :], x_bufs[0], in_sems[0])
    for k in range(NCHUNK):
        cur = k % 2
        r0 = base_row + k * RCHUNK
        if k + 1 < NCHUNK:
            in_dma[k + 1] = pltpu.async_copy(
                x_hbm.at[pl.ds(r0 + RCHUNK, RCHUNK), :],
                x_bufs[1 - cur], in_sems[1 - cur])
        in_dma[k].wait()
        if k >= 2:
            out_dma[k - 2].wait()
        x_v = x_bufs[cur]
        y_v = y_bufs[cur]

        @plsc.parallel_loop(0, RCHUNK, step=1, unroll=4)
        def _body(r):
            for c in COL_OFFS:
                v = x_v[r, pl.ds(c, L)]
                u = jnp.minimum(jnp.maximum(v * 15.0, 0.0), 15.0)
                idx = jnp.minimum(u, 14.0).astype(jnp.int32)
                av = _vgather(tab_a_vec, idx)
                bv = _vgather(tab_b_vec, idx)
                y_v[r, pl.ds(c, L)] = av + u * bv

        out_dma[k] = pltpu.async_copy(
            y_v, out_hbm.at[pl.ds(r0, RCHUNK), :], out_sems[cur])
    out_dma[NCHUNK - 2].wait()
    out_dma[NCHUNK - 1].wait()


def kernel(x, keypoint_y):
    return _calib(x, keypoint_y)


# speedup vs baseline: 1.0493x; 1.0185x over previous
import functools

import jax
import jax.numpy as jnp
from jax import lax
from jax.experimental import pallas as pl
from jax.experimental.pallas import tpu as pltpu
from jax.experimental.pallas import tpu_sc as plsc

NC, NS, L = 2, 16, 16
NW = NC * NS
N_KP = 16
ROWS, COLS = 16384, 100
ROWS_PER_TILE = ROWS // NW
RCHUNK = 128
NCHUNK = ROWS_PER_TILE // RCHUNK
COL_OFFS = (0, 16, 32, 48, 64, 80, 84)
MAGIC = 8388607.5


def _vgather(vec, idx):
    dn = lax.GatherDimensionNumbers(
        offset_dims=(), collapsed_slice_dims=(0,), start_index_map=(0,)
    )
    return lax.gather(
        vec, idx[:, None], dn, slice_sizes=(1,),
        mode=lax.GatherScatterMode.PROMISE_IN_BOUNDS,
    )


_mesh = plsc.VectorSubcoreMesh(
    core_axis_name="c", subcore_axis_name="s", num_cores=NC, num_subcores=NS
)


@functools.partial(
    pl.kernel,
    out_type=jax.ShapeDtypeStruct((ROWS, COLS), jnp.float32),
    mesh=_mesh,
    compiler_params=pltpu.CompilerParams(use_tc_tiling_on_sc=True),
    scratch_types=[
        pltpu.VMEM((RCHUNK, COLS), jnp.float32),
        pltpu.VMEM((RCHUNK, COLS), jnp.float32),
        pltpu.VMEM((RCHUNK, COLS), jnp.float32),
        pltpu.VMEM((RCHUNK, COLS), jnp.float32),
        pltpu.VMEM((RCHUNK, COLS), jnp.float32),
        pltpu.VMEM((RCHUNK, COLS), jnp.float32),
        pltpu.VMEM((N_KP,), jnp.float32),
        pltpu.SemaphoreType.DMA,
        pltpu.SemaphoreType.DMA,
        pltpu.SemaphoreType.DMA,
        pltpu.SemaphoreType.DMA,
        pltpu.SemaphoreType.DMA,
        pltpu.SemaphoreType.DMA,
    ],
)
def _calib(x_hbm, kp_hbm, out_hbm,
           x_v0, x_v1, x_v2, y_v0, y_v1, y_v2, kp_v,
           in_sem0, in_sem1, in_sem2, out_sem0, out_sem1, out_sem2):
    NBUF = 3
    x_bufs = (x_v0, x_v1, x_v2)
    y_bufs = (y_v0, y_v1, y_v2)
    in_sems = (in_sem0, in_sem1, in_sem2)
    out_sems = (out_sem0, out_sem1, out_sem2)
    wid = lax.axis_index("s") * NC + lax.axis_index("c")
    base_row = wid * ROWS_PER_TILE

    in_dma = [None] * NCHUNK
    out_dma = [None] * NCHUNK
    for k in range(min(NBUF - 1, NCHUNK)):
        in_dma[k] = pltpu.async_copy(
            x_hbm.at[pl.ds(base_row + k * RCHUNK, RCHUNK), :],
            x_bufs[k], in_sems[k])

    pltpu.sync_copy(kp_hbm, kp_v)

    lane = lax.iota(jnp.int32, L)
    lane_l = jnp.maximum(lane - 1, 0)
    raw = kp_v[...]
    y_r = 1.0 / (1.0 + jnp.exp(-raw))
    y_l = _vgather(y_r, lane_l)
    x_r = lane.astype(jnp.float32) * (1.0 / 15.0)
    x_l = lane_l.astype(jnp.float32) * (1.0 / 15.0)
    b_seg = (y_r - y_l) / (x_r - x_l + 1e-8)
    a_seg = y_l - x_l * b_seg
    shift = jnp.minimum(lane + 1, 15)
    tab_b_vec = _vgather(b_seg, shift) * (1.0 / 15.0)
    tab_a_vec = _vgather(a_seg, shift)

    for k in range(NCHUNK):
        cur = k % NBUF
        r0 = base_row + k * RCHUNK
        if k + NBUF - 1 < NCHUNK:
            nxt = (k + NBUF - 1) % NBUF
            in_dma[k + NBUF - 1] = pltpu.async_copy(
                x_hbm.at[pl.ds(r0 + (NBUF - 1) * RCHUNK, RCHUNK), :],
                x_bufs[nxt], in_sems[nxt])
        in_dma[k].wait()
        if k >= NBUF:
            out_dma[k - NBUF].wait()
        x_v = x_bufs[cur]
        y_v = y_bufs[cur]

        @plsc.parallel_loop(0, RCHUNK, step=1, unroll=4)
        def _body(r):
            for c in COL_OFFS:
                v = x_v[r, pl.ds(c, L)]
                u = jnp.minimum(jnp.maximum(v * 15.0, 0.0), 15.0)
                idx = jnp.minimum(u, 14.0).astype(jnp.int32)
                av = _vgather(tab_a_vec, idx)
                bv = _vgather(tab_b_vec, idx)
                y_v[r, pl.ds(c, L)] = av + u * bv

        out_dma[k] = pltpu.async_copy(
            y_v, out_hbm.at[pl.ds(r0, RCHUNK), :], out_sems[cur])
    for k in range(max(0, NCHUNK - NBUF), NCHUNK):
        out_dma[k].wait()


def kernel(x, keypoint_y):
    return _calib(x, keypoint_y)


# speedup vs baseline: 1.1275x; 1.0746x over previous
import functools

import jax
import jax.numpy as jnp
from jax import lax
from jax.experimental import pallas as pl
from jax.experimental.pallas import tpu as pltpu
from jax.experimental.pallas import tpu_sc as plsc

NC, NS, L = 2, 16, 16
NW = NC * NS
N_KP = 16
ROWS, COLS = 16384, 100
ROWS_PER_TILE = ROWS // NW
RCHUNK = 128
NCHUNK = ROWS_PER_TILE // RCHUNK
COL_OFFS = (0, 16, 32, 48, 64, 80, 84)
MAGIC = 8388607.5


def _vgather(vec, idx):
    dn = lax.GatherDimensionNumbers(
        offset_dims=(), collapsed_slice_dims=(0,), start_index_map=(0,)
    )
    return lax.gather(
        vec, idx[:, None], dn, slice_sizes=(1,),
        mode=lax.GatherScatterMode.PROMISE_IN_BOUNDS,
    )


_mesh = plsc.VectorSubcoreMesh(
    core_axis_name="c", subcore_axis_name="s", num_cores=NC, num_subcores=NS
)


@functools.partial(
    pl.kernel,
    out_type=jax.ShapeDtypeStruct((ROWS, COLS), jnp.float32),
    mesh=_mesh,
    compiler_params=pltpu.CompilerParams(
        use_tc_tiling_on_sc=True, needs_layout_passes=False
    ),
    scratch_types=[
        pltpu.VMEM((RCHUNK, COLS), jnp.float32),
        pltpu.VMEM((RCHUNK, COLS), jnp.float32),
        pltpu.VMEM((RCHUNK, COLS), jnp.float32),
        pltpu.VMEM((RCHUNK, COLS), jnp.float32),
        pltpu.VMEM((RCHUNK, COLS), jnp.float32),
        pltpu.VMEM((RCHUNK, COLS), jnp.float32),
        pltpu.VMEM((N_KP,), jnp.float32),
        pltpu.SemaphoreType.DMA,
        pltpu.SemaphoreType.DMA,
        pltpu.SemaphoreType.DMA,
        pltpu.SemaphoreType.DMA,
        pltpu.SemaphoreType.DMA,
        pltpu.SemaphoreType.DMA,
    ],
)
def _calib(x_hbm, kp_hbm, out_hbm,
           x_v0, x_v1, x_v2, y_v0, y_v1, y_v2, kp_v,
           in_sem0, in_sem1, in_sem2, out_sem0, out_sem1, out_sem2):
    NBUF = 3
    x_bufs = (x_v0, x_v1, x_v2)
    y_bufs = (y_v0, y_v1, y_v2)
    in_sems = (in_sem0, in_sem1, in_sem2)
    out_sems = (out_sem0, out_sem1, out_sem2)
    wid = lax.axis_index("s") * NC + lax.axis_index("c")
    base_row = wid * ROWS_PER_TILE

    in_dma = [None] * NCHUNK
    out_dma = [None] * NCHUNK
    for k in range(min(NBUF - 1, NCHUNK)):
        in_dma[k] = pltpu.async_copy(
            x_hbm.at[pl.ds(base_row + k * RCHUNK, RCHUNK), :],
            x_bufs[k], in_sems[k])

    pltpu.sync_copy(kp_hbm, kp_v)

    lane = lax.iota(jnp.int32, L)
    lane_l = jnp.maximum(lane - 1, 0)
    raw = kp_v[...]
    y_r = 1.0 / (1.0 + jnp.exp(-raw))
    y_l = _vgather(y_r, lane_l)
    x_r = lane.astype(jnp.float32) * (1.0 / 15.0)
    x_l = lane_l.astype(jnp.float32) * (1.0 / 15.0)
    b_seg = (y_r - y_l) / (x_r - x_l + 1e-8)
    a_seg = y_l - x_l * b_seg
    shift = jnp.minimum(lane + 1, 15)
    tab_b_vec = _vgather(b_seg, shift) * (1.0 / 15.0)
    tab_a_vec = _vgather(a_seg, shift)

    for k in range(NCHUNK):
        cur = k % NBUF
        r0 = base_row + k * RCHUNK
        if k + NBUF - 1 < NCHUNK:
            nxt = (k + NBUF - 1) % NBUF
            in_dma[k + NBUF - 1] = pltpu.async_copy(
                x_hbm.at[pl.ds(r0 + (NBUF - 1) * RCHUNK, RCHUNK), :],
                x_bufs[nxt], in_sems[nxt])
        in_dma[k].wait()
        if k >= NBUF:
            out_dma[k - NBUF].wait()
        x_v = x_bufs[cur]
        y_v = y_bufs[cur]

        @plsc.parallel_loop(0, RCHUNK, step=1, unroll=4)
        def _body(r):
            for c in COL_OFFS:
                v = x_v[r, pl.ds(c, L)]
                u = jnp.minimum(jnp.maximum(v * 15.0, 0.0), 15.0)
                idx = plsc.bitcast(u + MAGIC, jnp.int32) & 15
                av = _vgather(tab_a_vec, idx)
                bv = _vgather(tab_b_vec, idx)
                y_v[r, pl.ds(c, L)] = av + u * bv

        out_dma[k] = pltpu.async_copy(
            y_v, out_hbm.at[pl.ds(r0, RCHUNK), :], out_sems[cur])
    for k in range(max(0, NCHUNK - NBUF), NCHUNK):
        out_dma[k].wait()


def kernel(x, keypoint_y):
    return _calib(x, keypoint_y)
